# baseline (speedup 1.0000x reference)
import jax
import jax.numpy as jnp
from jax.experimental import pallas as pl

TOK = 512


def _gating_body(x_ref, w_ref, b_ref, o_ref):
    xh = x_ref[...].astype(jnp.bfloat16)
    wh = w_ref[...].astype(jnp.bfloat16)
    logits = jnp.dot(xh, wh, preferred_element_type=jnp.float32)
    logits = logits + b_ref[...]
    m = jnp.max(logits, axis=-1, keepdims=True)
    e = jnp.exp(logits - m)
    o_ref[...] = e / jnp.sum(e, axis=-1, keepdims=True)


def kernel(x, W, b):
    B, S, D = x.shape
    E = W.shape[1]
    N = B * S
    xf = x.reshape(N, D)
    b2 = b.reshape(1, E)

    out = pl.pallas_call(
        _gating_body,
        grid=(N // TOK,),
        in_specs=[
            pl.BlockSpec((TOK, D), lambda i: (i, 0)),
            pl.BlockSpec((D, E), lambda i: (0, 0)),
            pl.BlockSpec((1, E), lambda i: (0, 0)),
        ],
        out_specs=pl.BlockSpec((TOK, E), lambda i: (i, 0)),
        out_shape=jax.ShapeDtypeStruct((N, E), jnp.float32),
    )(xf, W, b2)
    return out.reshape(B, S, E)


# speedup vs baseline: 1.0165x; 1.0165x over previous
import jax
import jax.numpy as jnp
from jax.experimental import pallas as pl

TOK = 1024


def _gating_body(x_ref, w_ref, b_ref, o_ref):
    xh = x_ref[...].astype(jnp.bfloat16)
    wh = w_ref[...].astype(jnp.bfloat16)
    logits = jnp.dot(xh, wh, preferred_element_type=jnp.float32)
    logits = logits + b_ref[...]
    m = jnp.max(logits, axis=-1, keepdims=True)
    e = jnp.exp(logits - m)
    o_ref[...] = e / jnp.sum(e, axis=-1, keepdims=True)


def kernel(x, W, b):
    B, S, D = x.shape
    E = W.shape[1]
    N = B * S
    xf = x.reshape(N, D)
    b2 = b.reshape(1, E)

    out = pl.pallas_call(
        _gating_body,
        grid=(N // TOK,),
        in_specs=[
            pl.BlockSpec((TOK, D), lambda i: (i, 0)),
            pl.BlockSpec((D, E), lambda i: (0, 0)),
            pl.BlockSpec((1, E), lambda i: (0, 0)),
        ],
        out_specs=pl.BlockSpec((TOK, E), lambda i: (i, 0)),
        out_shape=jax.ShapeDtypeStruct((N, E), jnp.float32),
    )(xf, W, b2)
    return out.reshape(B, S, E)
